# Initial kernel scaffold; baseline (speedup 1.0000x reference)
import jax
import jax.numpy as jnp
from jax.experimental import pallas as pl


def kernel(input_ids, token_type_ids, word_embedding, token_type_table, position_embedding, ln_gamma, ln_beta):
    raise NotImplementedError("write your pallas kernel here")


# speedup vs baseline: 1.4491x; 1.4491x over previous
import functools

import jax
import jax.numpy as jnp
from jax import lax
from jax.experimental import pallas as pl
from jax.experimental.pallas import tpu as pltpu
from jax.experimental.pallas import tpu_sc as plsc

_B, _S, _D = 4, 2048, 1024
_N_TOK = _B * _S
_NC, _NS = 2, 16
_NW = _NC * _NS
_PER_W = _N_TOK // _NW
_CHUNK = 64


def _sc_gather(table, idx):
    mesh = plsc.VectorSubcoreMesh(core_axis_name="c", subcore_axis_name="s")

    @functools.partial(
        pl.kernel,
        mesh=mesh,
        out_type=jax.ShapeDtypeStruct((_N_TOK, _D), jnp.float32),
        scratch_types=[
            pltpu.VMEM((_CHUNK,), jnp.int32),
            pltpu.VMEM((_CHUNK, _D), jnp.float32),
            pltpu.SemaphoreType.DMA,
        ],
    )
    def k(table_hbm, idx_hbm, out_hbm, idx_v, rows_v, sem):
        wid = lax.axis_index("s") * _NC + lax.axis_index("c")
        base = wid * _PER_W
        for j in range(_PER_W // _CHUNK):
            off = base + j * _CHUNK
            pltpu.sync_copy(idx_hbm.at[pl.ds(off, _CHUNK)], idx_v)
            pltpu.async_copy(table_hbm.at[idx_v], rows_v, sem).wait()
            pltpu.sync_copy(rows_v, out_hbm.at[pl.ds(off, _CHUNK)])

    return k(table, idx)


def _tc_body(g_ref, pos_ref, ttf_ref, ttab_ref, gam_ref, bet_ref, o_ref):
    x = g_ref[...]
    ttf = ttf_ref[...]
    t0 = ttab_ref[0:1, :]
    t1 = ttab_ref[1:2, :]
    x = x + pos_ref[...] + t0 + ttf * (t1 - t0)
    mean = jnp.mean(x, axis=1, keepdims=True)
    xc = x - mean
    var = jnp.mean(xc * xc, axis=1, keepdims=True)
    y = xc * lax.rsqrt(var + 1e-12)
    o_ref[...] = y * gam_ref[...] + bet_ref[...]


_ROWS = 512


def _tc_ln(gathered, pos, ttf, ttab, gamma, beta):
    n_s = _S // _ROWS
    return pl.pallas_call(
        _tc_body,
        grid=(n_s, _B),
        in_specs=[
            pl.BlockSpec((_ROWS, _D), lambda i, b: (b * n_s + i, 0)),
            pl.BlockSpec((_ROWS, _D), lambda i, b: (i, 0)),
            pl.BlockSpec((_ROWS, 1), lambda i, b: (b * n_s + i, 0)),
            pl.BlockSpec((2, _D), lambda i, b: (0, 0)),
            pl.BlockSpec((1, _D), lambda i, b: (0, 0)),
            pl.BlockSpec((1, _D), lambda i, b: (0, 0)),
        ],
        out_specs=pl.BlockSpec((_ROWS, _D), lambda i, b: (b * n_s + i, 0)),
        out_shape=jax.ShapeDtypeStruct((_N_TOK, _D), jnp.float32),
    )(gathered, pos, ttf, ttab, gamma, beta)


def kernel(input_ids, token_type_ids, word_embedding, token_type_table,
           position_embedding, ln_gamma, ln_beta):
    flat_ids = input_ids.reshape(-1).astype(jnp.int32)
    gathered = _sc_gather(word_embedding, flat_ids)
    ttf = token_type_ids.reshape(-1, 1).astype(jnp.float32)
    out = _tc_ln(gathered, position_embedding, ttf, token_type_table,
                 ln_gamma.reshape(1, _D), ln_beta.reshape(1, _D))
    return out.reshape(_B, _S, _D), word_embedding


# speedup vs baseline: 1.5026x; 1.0369x over previous
import functools

import jax
import jax.numpy as jnp
from jax import lax
from jax.experimental import pallas as pl
from jax.experimental.pallas import tpu as pltpu
from jax.experimental.pallas import tpu_sc as plsc

_B, _S, _D = 4, 2048, 1024
_N_TOK = _B * _S
_NC, _NS = 2, 16
_NW = _NC * _NS
_PER_W = _N_TOK // _NW
_CHUNK = 64


def _sc_gather(table, idx):
    mesh = plsc.VectorSubcoreMesh(core_axis_name="c", subcore_axis_name="s")

    @functools.partial(
        pl.kernel,
        mesh=mesh,
        out_type=jax.ShapeDtypeStruct((_N_TOK, _D), jnp.float32),
        scratch_types=[
            pltpu.VMEM((_CHUNK,), jnp.int32),
            pltpu.VMEM((_CHUNK, _D), jnp.float32),
            pltpu.SemaphoreType.DMA,
        ],
    )
    def k(table_hbm, idx_hbm, out_hbm, idx_v, rows_v, sem):
        wid = lax.axis_index("s") * _NC + lax.axis_index("c")
        base = wid * _PER_W
        for j in range(_PER_W // _CHUNK):
            off = base + j * _CHUNK
            pltpu.sync_copy(idx_hbm.at[pl.ds(off, _CHUNK)], idx_v)
            pltpu.async_copy(table_hbm.at[idx_v], rows_v, sem).wait()
            pltpu.sync_copy(rows_v, out_hbm.at[pl.ds(off, _CHUNK)])

    return k(table, idx)


def _copy_body(w_ref, o_ref):
    o_ref[...] = w_ref[...]


_V = 30522
_CP_ROWS = 2048


def _tc_table_copy(table):
    grid = (_V + _CP_ROWS - 1) // _CP_ROWS
    return pl.pallas_call(
        _copy_body,
        grid=(grid,),
        in_specs=[pl.BlockSpec((_CP_ROWS, _D), lambda i: (i, 0))],
        out_specs=pl.BlockSpec((_CP_ROWS, _D), lambda i: (i, 0)),
        out_shape=jax.ShapeDtypeStruct((_V, _D), jnp.float32),
    )(table)


def _tc_body(g_ref, pos_ref, ttf_ref, ttab_ref, gam_ref, bet_ref, w_ref, o_ref):
    x = g_ref[...]
    ttf = ttf_ref[...]
    t0 = ttab_ref[0:1, :]
    t1 = ttab_ref[1:2, :]
    x = x + pos_ref[...] + t0 + ttf * (t1 - t0)
    mean = jnp.mean(x, axis=1, keepdims=True)
    xc = x - mean
    var = jnp.mean(xc * xc, axis=1, keepdims=True)
    y = xc * lax.rsqrt(var + 1e-12)
    o_ref[...] = y * gam_ref[...] + bet_ref[...]


_ROWS = 512


def _tc_ln(gathered, pos, ttf, ttab, gamma, beta, wout):
    n_s = _S // _ROWS
    return pl.pallas_call(
        _tc_body,
        grid=(n_s, _B),
        in_specs=[
            pl.BlockSpec((_ROWS, _D), lambda i, b: (b * n_s + i, 0)),
            pl.BlockSpec((_ROWS, _D), lambda i, b: (i, 0)),
            pl.BlockSpec((_ROWS, 1), lambda i, b: (b * n_s + i, 0)),
            pl.BlockSpec((2, _D), lambda i, b: (0, 0)),
            pl.BlockSpec((1, _D), lambda i, b: (0, 0)),
            pl.BlockSpec((1, _D), lambda i, b: (0, 0)),
            pl.BlockSpec((8, 128), lambda i, b: (0, 0)),
        ],
        out_specs=pl.BlockSpec((_ROWS, _D), lambda i, b: (b * n_s + i, 0)),
        out_shape=jax.ShapeDtypeStruct((_N_TOK, _D), jnp.float32),
    )(gathered, pos, ttf, ttab, gamma, beta, wout)


def kernel(input_ids, token_type_ids, word_embedding, token_type_table,
           position_embedding, ln_gamma, ln_beta):
    flat_ids = input_ids.reshape(-1).astype(jnp.int32)
    gathered = _sc_gather(word_embedding, flat_ids)
    wout = _tc_table_copy(word_embedding)
    ttf = token_type_ids.reshape(-1, 1).astype(jnp.float32)
    out = _tc_ln(gathered, position_embedding, ttf, token_type_table,
                 ln_gamma.reshape(1, _D), ln_beta.reshape(1, _D), wout)
    return out.reshape(_B, _S, _D), wout
